# baseline (speedup 1.0000x reference)
import functools

import jax
import jax.numpy as jnp
from jax import lax
from jax.experimental import pallas as pl
from jax.experimental.pallas import tpu as pltpu
from jax.experimental.pallas import tpu_sc as plsc

B = 1024
L = 20
D = 64
NV = 1001
V = 1024
NBT = 10
SO2 = 2
GW = 128
BLK = 16
ALPHA = 0.2
KCONST = 12.0

_f32 = jnp.float32
_HI = lax.Precision.HIGHEST


def _dot(a, b):
    return jnp.dot(a, b, preferred_element_type=_f32, precision=_HI)


def _dotd(a, b):
    return jnp.dot(a.astype(jnp.bfloat16), b.astype(jnp.bfloat16),
                   preferred_element_type=_f32)


def _leaky(x):
    return jnp.where(x >= 0, x, ALPHA * x)


def _prep_body(emb_ref, w1r_ref, w2r_ref, w1l_ref, w2l_ref, posf_ref, posb_ref,
               w1b_ref, w2b_ref, w4r_ref, w4b_ref,
               table_ref, bias4_ref):
    emb = emb_ref[...]
    zpad = jnp.zeros((V, GW - D), _f32)
    table_ref[0] = jnp.concatenate([emb, emb], axis=-1)

    def branch_tables(wr_ref, pos_ref, wl_ref, wb_ref, slot):
        t = _dot(emb, wr_ref[...])
        pw = _dot(pos_ref[0:8, :], wl_ref[...])
        b0 = pw[0:1, :] + wb_ref[...]
        b1 = pw[1:2, :] + wb_ref[...]
        table_ref[slot] = jnp.concatenate([jnp.tanh(t + b0), zpad], axis=-1)
        table_ref[slot + 1] = jnp.concatenate([zpad, jnp.tanh(t + b1)], axis=-1)

    branch_tables(w1r_ref, posf_ref, w1l_ref, w1b_ref, 1)
    branch_tables(w2r_ref, posb_ref, w2l_ref, w2b_ref, 3)

    p4 = _dot(posf_ref[0:24, :], w4r_ref[...])
    b4 = p4[0:L, :] + w4b_ref[...]
    bias4_ref[...] = jnp.concatenate([b4, jnp.zeros((32 - L, D), _f32)], axis=0)


_prep = pl.pallas_call(
    _prep_body,
    out_shape=(
        jax.ShapeDtypeStruct((5, V, GW), _f32),
        jax.ShapeDtypeStruct((32, D), _f32),
    ),
)


_WIN = 128


def _sc_gather(table, idx):
    n = idx.shape[0]
    idx2 = idx.reshape(1, n)
    mesh = plsc.VectorSubcoreMesh(core_axis_name="core",
                                  subcore_axis_name="subcore")

    @functools.partial(
        pl.kernel,
        out_type=jax.ShapeDtypeStruct((n, GW), _f32),
        mesh=mesh,
    )
    def k(table_hbm, i_hbm, o_hbm):
        def body(i_vmem, o_vmem):
            pltpu.sync_copy(table_hbm.at[i_vmem.at[0]], o_vmem)

        pltpu.emit_pipeline(
            body,
            grid=(n // _WIN,),
            in_specs=[pl.BlockSpec((1, _WIN), index_map=lambda i: (0, i))],
            out_specs=[pl.BlockSpec((_WIN, GW), index_map=lambda i: (i, 0))],
            core_axis_name=("core", "subcore"),
            dimension_semantics=(pltpu.PARALLEL,),
        )(i_hbm, o_hbm)

    return k(table, idx2)


NB5 = 5


def _gat(tar128, raw_ref, w_ref, qblk, qb128, rmat, emat):
    blk_l = BLK * L
    raw = raw_ref[...].reshape(blk_l, 2, NB5, GW)
    ne = raw[:, 0] + raw[:, 1]
    wp = w_ref[...]
    wp3 = wp.reshape(blk_l, NB5, 2)
    wlane = _dotd(wp, emat).reshape(blk_l, NB5, GW)
    x = tar128[:, None, :] * ne
    ap = _dotd(x.reshape(blk_l * NB5, GW), qblk).reshape(blk_l, NB5, GW)
    ap = ap + wlane * qb128[None]
    asc = _dotd(_leaky(ap).reshape(blk_l * NB5, GW), rmat)[:, 0:2]
    asc = asc.reshape(blk_l, NB5, 2)
    asc = asc + jnp.where(wp3 != 0.0, 0.0, -10000.0)
    mx = jnp.max(jnp.max(asc, axis=2, keepdims=True), axis=1, keepdims=True)
    e = jnp.exp(asc - mx)
    den = jnp.sum(jnp.sum(e, axis=2, keepdims=True), axis=1, keepdims=True)
    alpha = (e / den) * (wp3 != 0.0).astype(_f32)
    al_lane = _dotd(alpha.reshape(blk_l * NB5, 2), emat).reshape(blk_l, NB5, GW)
    prod = jnp.sum(al_lane * ne, axis=1)
    return prod[:, 0:D] + prod[:, D:GW]


def _main_body(gf_ref, gb_ref, gt_ref, wf_ref, wb_ref, al_ref,
               b4_ref, mats_ref, misc_ref, fc1b_ref, embt_ref,
               qbf_ref, qbb_ref, rf_ref, rb_ref, em_ref,
               sp_ref, sd_ref):
    blk_l = BLK * L
    tar128 = gt_ref[...]
    tar = tar128[:, 0:D]
    emat = em_ref[0:2, :]

    nf = _gat(tar128, gf_ref, wf_ref, qbf_ref[...], misc_ref[0:1, :],
              rf_ref[...], emat)
    nb = _gat(tar128, gb_ref, wb_ref, qbb_ref[...], misc_ref[1:2, :],
              rb_ref[...], emat)
    neig = nf + nb

    gate = jax.nn.sigmoid(_dot(neig, mats_ref[0]) + _dot(tar, mats_ref[1])
                          + misc_ref[2:3, 0:D])
    fin = gate * neig + (1.0 - gate) * tar
    fin3 = fin.reshape(BLK, L, D)

    al = al_ref[...]
    oh = (al[:, :, None]
          == lax.broadcasted_iota(jnp.int32, (BLK, L, L), 2)).astype(_f32)
    af = jnp.sum(oh[:, :, :, None] * fin3[:, None, :, :], axis=2)

    fp = jnp.tanh(_dot(af.reshape(blk_l, D), mats_ref[2]).reshape(BLK, L, D)
                  + b4_ref[0:L, :][None])
    a5 = jnp.sum(_leaky(fp) * misc_ref[3:4, 0:D][None], axis=-1)
    a5 = a5 - jnp.max(a5, axis=-1, keepdims=True)
    e5 = jnp.exp(a5)
    a5 = e5 / jnp.sum(e5, axis=-1, keepdims=True)
    sess = jnp.sum(a5[:, :, None] * fp, axis=1)

    mu = jnp.mean(sess, axis=-1, keepdims=True)
    var = jnp.mean((sess - mu) ** 2, axis=-1, keepdims=True)
    sess = (sess - mu) / jnp.sqrt(var + 1e-5) * misc_ref[4:5, 0:D] \
        + misc_ref[5:6, 0:D]

    embt = embt_ref[...]
    sp = _dot(sess, embt)
    h = jnp.maximum(_dot(mats_ref[3], embt) + fc1b_ref[:, 0:1], 0.0)
    y = _dot(misc_ref[6:7, 0:D], h) + misc_ref[7:8, 0:1]
    s = jax.nn.sigmoid(y)
    sp_ref[...] = sp[:, :NV]
    sd_ref[...] = (sp * s - KCONST * s)[:, :NV]


def _main(gathered, wf2, wb2, alias, bias4, mats, misc, fc1bc, embt,
          qbf, qbb, rf, rb, em):
    nblk = B // BLK
    off_b = (B * L * NBT) // (BLK * L * NBT)
    off_t = (2 * B * L * NBT) // (BLK * L)
    grid = (nblk,)
    full = lambda shape: pl.BlockSpec(shape, lambda i: (0, 0))
    full3 = lambda shape: pl.BlockSpec(shape, lambda i: (0, 0, 0))
    return pl.pallas_call(
        _main_body,
        grid=grid,
        in_specs=[
            pl.BlockSpec((BLK * L * NBT, GW), lambda i: (i, 0)),
            pl.BlockSpec((BLK * L * NBT, GW), lambda i, o=off_b: (i + o, 0)),
            pl.BlockSpec((BLK * L, GW), lambda i, o=off_t: (i + o, 0)),
            pl.BlockSpec((BLK * L * NB5, 2), lambda i: (i, 0)),
            pl.BlockSpec((BLK * L * NB5, 2), lambda i: (i, 0)),
            pl.BlockSpec((BLK, L), lambda i: (i, 0)),
            full((32, D)),
            full3((4, D, D)),
            full((16, GW)),
            full((D, 8)),
            full((D, V)),
            full((GW, GW)),
            full((GW, GW)),
            full((GW, 8)),
            full((GW, 8)),
            full((8, GW)),
        ],
        out_specs=(
            pl.BlockSpec((BLK, NV), lambda i: (i, 0)),
            pl.BlockSpec((BLK, NV), lambda i: (i, 0)),
        ),
        out_shape=(
            jax.ShapeDtypeStruct((B, NV), _f32),
            jax.ShapeDtypeStruct((B, NV), _f32),
        ),
    )(gathered, gathered, gathered, wf2, wb2, alias, bias4,
      mats, misc, fc1bc, embt, qbf, qbb, rf, rb, em)


def kernel(alias_re_inputs, items, mask, f_adjacency_nodes, f_adjacency_weight,
           b_adjacency_nodes, b_adjacency_weight, emb_w, f_pos, b_pos,
           W1_w, W1_b, q1, q2, W2_w, W2_b, q3, q4, W3_w, W3_b,
           W4_w, W4_b, q5, ln_g, ln_b, fc1_w, fc1_b, fc2_w, fc2_b):
    emb_pad = jnp.zeros((V, D), _f32).at[1:NV].set(emb_w)

    table, bias4 = _prep(
        emb_pad, W1_w[:, D:].T, W2_w[:, D:].T, W1_w[:, :D].T, W2_w[:, :D].T,
        f_pos, b_pos, W1_b.reshape(1, D), W2_b.reshape(1, D),
        W4_w[:, D:].T, W4_b.reshape(1, D))

    so_off = (jnp.arange(2, dtype=jnp.int32) * V).reshape(1, 1, 2, 1)
    idx_f = (f_adjacency_nodes.astype(jnp.int32) + V + so_off).reshape(-1)
    idx_b = (b_adjacency_nodes.astype(jnp.int32) + 3 * V + so_off).reshape(-1)
    idx_t = items.reshape(-1).astype(jnp.int32)
    idx_all = jnp.concatenate([idx_f, idx_b, idx_t])

    gathered = _sc_gather(table.reshape(5 * V, GW), idx_all)

    wf2 = f_adjacency_weight.transpose(0, 1, 3, 2).reshape(B * L * NB5, SO2)
    wb2 = b_adjacency_weight.transpose(0, 1, 3, 2).reshape(B * L * NB5, SO2)

    mats = jnp.stack([W3_w[:, :D].T, W3_w[:, D:].T, W4_w[:, :D].T, fc1_w])

    z64 = jnp.zeros((1, D), _f32)
    misc = jnp.concatenate([
        jnp.concatenate([q1[D:], q1[D:]], axis=1),
        jnp.concatenate([q3[D:], q3[D:]], axis=1),
        jnp.concatenate([W3_b[None], z64], axis=1),
        jnp.concatenate([q5.T, z64], axis=1),
        jnp.concatenate([ln_g[None], z64], axis=1),
        jnp.concatenate([ln_b[None], z64], axis=1),
        jnp.concatenate([fc2_w, z64], axis=1),
        jnp.zeros((1, GW), _f32) + fc2_b[0],
    ], axis=0)
    misc = jnp.pad(misc, ((0, 8), (0, 0)))
    fc1bc = jnp.broadcast_to(fc1_b.reshape(D, 1), (D, 8))
    embt = emb_pad.T

    zdd = jnp.zeros((D, D), _f32)
    qbf = jnp.concatenate([
        jnp.concatenate([q1[:D], zdd], axis=1),
        jnp.concatenate([zdd, q1[:D]], axis=1)], axis=0)
    qbb = jnp.concatenate([
        jnp.concatenate([q3[:D], zdd], axis=1),
        jnp.concatenate([zdd, q3[:D]], axis=1)], axis=0)
    zc = jnp.zeros((D, 1), _f32)
    rf = jnp.concatenate([
        jnp.concatenate([q2, zc], axis=1),
        jnp.concatenate([zc, q2], axis=1)], axis=0)
    rf = jnp.pad(rf, ((0, 0), (0, 6)))
    rb = jnp.concatenate([
        jnp.concatenate([q4, zc], axis=1),
        jnp.concatenate([zc, q4], axis=1)], axis=0)
    rb = jnp.pad(rb, ((0, 0), (0, 6)))
    ones64 = jnp.ones((1, D), _f32)
    em = jnp.concatenate([
        jnp.concatenate([ones64, z64], axis=1),
        jnp.concatenate([z64, ones64], axis=1)], axis=0)
    em = jnp.pad(em, ((0, 6), (0, 0)))

    return _main(gathered, wf2, wb2, alias_re_inputs.astype(jnp.int32),
                 bias4, mats, misc, fc1bc, embt, qbf, qbb, rf, rb, em)


# speedup vs baseline: 1.0747x; 1.0747x over previous
import functools

import jax
import jax.numpy as jnp
from jax import lax
from jax.experimental import pallas as pl
from jax.experimental.pallas import tpu as pltpu
from jax.experimental.pallas import tpu_sc as plsc

B = 1024
L = 20
D = 64
NV = 1001
V = 1024
NBT = 10
SO2 = 2
GW = 128
BLK = 16
ALPHA = 0.2
KCONST = 12.0

_f32 = jnp.float32
_HI = lax.Precision.HIGHEST


def _dot(a, b):
    return jnp.dot(a, b, preferred_element_type=_f32, precision=_HI)


def _dotd(a, b):
    return jnp.dot(a.astype(jnp.bfloat16), b.astype(jnp.bfloat16),
                   preferred_element_type=_f32)


def _leaky(x):
    return jnp.where(x >= 0, x, ALPHA * x)


def _prep_body(emb_ref, w1r_ref, w2r_ref, w1l_ref, w2l_ref, posf_ref, posb_ref,
               w1b_ref, w2b_ref, w4r_ref, w4b_ref,
               table_ref, bias4_ref):
    emb = emb_ref[...]
    zpad = jnp.zeros((V, GW - D), _f32)
    table_ref[0] = jnp.concatenate([emb, emb], axis=-1)

    def branch_tables(wr_ref, pos_ref, wl_ref, wb_ref, slot):
        t = _dot(emb, wr_ref[...])
        pw = _dot(pos_ref[0:8, :], wl_ref[...])
        b0 = pw[0:1, :] + wb_ref[...]
        b1 = pw[1:2, :] + wb_ref[...]
        table_ref[slot] = jnp.concatenate([jnp.tanh(t + b0), zpad], axis=-1)
        table_ref[slot + 1] = jnp.concatenate([zpad, jnp.tanh(t + b1)], axis=-1)

    branch_tables(w1r_ref, posf_ref, w1l_ref, w1b_ref, 1)
    branch_tables(w2r_ref, posb_ref, w2l_ref, w2b_ref, 3)

    p4 = _dot(posf_ref[0:24, :], w4r_ref[...])
    b4 = p4[0:L, :] + w4b_ref[...]
    bias4_ref[...] = jnp.concatenate([b4, jnp.zeros((32 - L, D), _f32)], axis=0)


_prep = pl.pallas_call(
    _prep_body,
    out_shape=(
        jax.ShapeDtypeStruct((5, V, GW), _f32),
        jax.ShapeDtypeStruct((32, D), _f32),
    ),
)


_WIN = 128


def _sc_gather(table, idx):
    n = idx.shape[0]
    idx2 = idx.reshape(1, n)
    mesh = plsc.VectorSubcoreMesh(core_axis_name="core",
                                  subcore_axis_name="subcore")

    @functools.partial(
        pl.kernel,
        out_type=jax.ShapeDtypeStruct((n, GW), _f32),
        mesh=mesh,
    )
    def k(table_hbm, i_hbm, o_hbm):
        def body(i_vmem, o_vmem):
            pltpu.sync_copy(table_hbm.at[i_vmem.at[0]], o_vmem)

        pltpu.emit_pipeline(
            body,
            grid=(n // _WIN,),
            in_specs=[pl.BlockSpec((1, _WIN), index_map=lambda i: (0, i))],
            out_specs=[pl.BlockSpec((_WIN, GW), index_map=lambda i: (i, 0))],
            core_axis_name=("core", "subcore"),
            dimension_semantics=(pltpu.PARALLEL,),
        )(i_hbm, o_hbm)

    return k(table, idx2)


NB5 = 5


def _gat(tar128, r0_ref, r1_ref, w_ref, qblk, qb128, rmat, emat):
    blk_l = BLK * L
    ne = (r0_ref[...] + r1_ref[...]).reshape(blk_l, NB5, GW)
    wp = w_ref[...]
    wp3 = wp.reshape(blk_l, NB5, 2)
    wlane = _dotd(wp, emat).reshape(blk_l, NB5, GW)
    x = tar128[:, None, :] * ne
    ap = _dotd(x.reshape(blk_l * NB5, GW), qblk).reshape(blk_l, NB5, GW)
    ap = ap + wlane * qb128[None]
    asc = _dotd(_leaky(ap).reshape(blk_l * NB5, GW), rmat)[:, 0:2]
    asc = asc.reshape(blk_l, NB5, 2)
    asc = asc + jnp.where(wp3 != 0.0, 0.0, -10000.0)
    mx = jnp.max(jnp.max(asc, axis=2, keepdims=True), axis=1, keepdims=True)
    e = jnp.exp(asc - mx)
    den = jnp.sum(jnp.sum(e, axis=2, keepdims=True), axis=1, keepdims=True)
    alpha = (e / den) * (wp3 != 0.0).astype(_f32)
    al_lane = _dotd(alpha.reshape(blk_l * NB5, 2), emat).reshape(blk_l, NB5, GW)
    prod = jnp.sum(al_lane * ne, axis=1)
    return prod[:, 0:D] + prod[:, D:GW]


def _main_body(gf0_ref, gf1_ref, gb0_ref, gb1_ref, gt_ref, wf_ref, wb_ref,
               al_ref, b4_ref, mats_ref, misc_ref, fc1b_ref, embt_ref,
               qbf_ref, qbb_ref, rf_ref, rb_ref, em_ref,
               sp_ref, sd_ref):
    blk_l = BLK * L
    tar128 = gt_ref[...]
    tar = tar128[:, 0:D]
    emat = em_ref[0:2, :]

    nf = _gat(tar128, gf0_ref, gf1_ref, wf_ref, qbf_ref[...],
              misc_ref[0:1, :], rf_ref[...], emat)
    nb = _gat(tar128, gb0_ref, gb1_ref, wb_ref, qbb_ref[...],
              misc_ref[1:2, :], rb_ref[...], emat)
    neig = nf + nb

    gate = jax.nn.sigmoid(_dot(neig, mats_ref[0]) + _dot(tar, mats_ref[1])
                          + misc_ref[2:3, 0:D])
    fin = gate * neig + (1.0 - gate) * tar
    fin3 = fin.reshape(BLK, L, D)

    al = al_ref[...]
    oh = (al[:, :, None]
          == lax.broadcasted_iota(jnp.int32, (BLK, L, L), 2)).astype(_f32)
    af = jnp.sum(oh[:, :, :, None] * fin3[:, None, :, :], axis=2)

    fp = jnp.tanh(_dot(af.reshape(blk_l, D), mats_ref[2]).reshape(BLK, L, D)
                  + b4_ref[0:L, :][None])
    a5 = jnp.sum(_leaky(fp) * misc_ref[3:4, 0:D][None], axis=-1)
    a5 = a5 - jnp.max(a5, axis=-1, keepdims=True)
    e5 = jnp.exp(a5)
    a5 = e5 / jnp.sum(e5, axis=-1, keepdims=True)
    sess = jnp.sum(a5[:, :, None] * fp, axis=1)

    mu = jnp.mean(sess, axis=-1, keepdims=True)
    var = jnp.mean((sess - mu) ** 2, axis=-1, keepdims=True)
    sess = (sess - mu) / jnp.sqrt(var + 1e-5) * misc_ref[4:5, 0:D] \
        + misc_ref[5:6, 0:D]

    embt = embt_ref[...]
    sp = _dot(sess, embt)
    h = jnp.maximum(_dot(mats_ref[3], embt) + fc1b_ref[:, 0:1], 0.0)
    y = _dot(misc_ref[6:7, 0:D], h) + misc_ref[7:8, 0:1]
    s = jax.nn.sigmoid(y)
    sp_ref[...] = sp[:, :NV]
    sd_ref[...] = (sp * s - KCONST * s)[:, :NV]


def _main(gathered, wf2, wb2, alias, bias4, mats, misc, fc1bc, embt,
          qbf, qbb, rf, rb, em):
    nblk = B // BLK
    seg = B * L * NB5
    rblk = BLK * L * NB5
    o1, o2, o3 = seg // rblk, 2 * seg // rblk, 3 * seg // rblk
    off_t = (4 * seg) // (BLK * L)
    grid = (nblk,)
    full = lambda shape: pl.BlockSpec(shape, lambda i: (0, 0))
    full3 = lambda shape: pl.BlockSpec(shape, lambda i: (0, 0, 0))
    return pl.pallas_call(
        _main_body,
        grid=grid,
        in_specs=[
            pl.BlockSpec((rblk, GW), lambda i: (i, 0)),
            pl.BlockSpec((rblk, GW), lambda i, o=o1: (i + o, 0)),
            pl.BlockSpec((rblk, GW), lambda i, o=o2: (i + o, 0)),
            pl.BlockSpec((rblk, GW), lambda i, o=o3: (i + o, 0)),
            pl.BlockSpec((BLK * L, GW), lambda i, o=off_t: (i + o, 0)),
            pl.BlockSpec((BLK * L * NB5, 2), lambda i: (i, 0)),
            pl.BlockSpec((BLK * L * NB5, 2), lambda i: (i, 0)),
            pl.BlockSpec((BLK, L), lambda i: (i, 0)),
            full((32, D)),
            full3((4, D, D)),
            full((16, GW)),
            full((D, 8)),
            full((D, V)),
            full((GW, GW)),
            full((GW, GW)),
            full((GW, 8)),
            full((GW, 8)),
            full((8, GW)),
        ],
        out_specs=(
            pl.BlockSpec((BLK, NV), lambda i: (i, 0)),
            pl.BlockSpec((BLK, NV), lambda i: (i, 0)),
        ),
        out_shape=(
            jax.ShapeDtypeStruct((B, NV), _f32),
            jax.ShapeDtypeStruct((B, NV), _f32),
        ),
    )(gathered, gathered, gathered, gathered, gathered, wf2, wb2, alias,
      bias4, mats, misc, fc1bc, embt, qbf, qbb, rf, rb, em)


def kernel(alias_re_inputs, items, mask, f_adjacency_nodes, f_adjacency_weight,
           b_adjacency_nodes, b_adjacency_weight, emb_w, f_pos, b_pos,
           W1_w, W1_b, q1, q2, W2_w, W2_b, q3, q4, W3_w, W3_b,
           W4_w, W4_b, q5, ln_g, ln_b, fc1_w, fc1_b, fc2_w, fc2_b):
    emb_pad = jnp.zeros((V, D), _f32).at[1:NV].set(emb_w)

    table, bias4 = _prep(
        emb_pad, W1_w[:, D:].T, W2_w[:, D:].T, W1_w[:, :D].T, W2_w[:, :D].T,
        f_pos, b_pos, W1_b.reshape(1, D), W2_b.reshape(1, D),
        W4_w[:, D:].T, W4_b.reshape(1, D))

    fn = f_adjacency_nodes.astype(jnp.int32)
    bn = b_adjacency_nodes.astype(jnp.int32)
    idx_all = jnp.concatenate([
        fn[:, :, 0, :].reshape(-1) + V,
        fn[:, :, 1, :].reshape(-1) + 2 * V,
        bn[:, :, 0, :].reshape(-1) + 3 * V,
        bn[:, :, 1, :].reshape(-1) + 4 * V,
        items.reshape(-1).astype(jnp.int32),
    ])

    gathered = _sc_gather(table.reshape(5 * V, GW), idx_all)

    wf2 = f_adjacency_weight.transpose(0, 1, 3, 2).reshape(B * L * NB5, SO2)
    wb2 = b_adjacency_weight.transpose(0, 1, 3, 2).reshape(B * L * NB5, SO2)

    mats = jnp.stack([W3_w[:, :D].T, W3_w[:, D:].T, W4_w[:, :D].T, fc1_w])

    z64 = jnp.zeros((1, D), _f32)
    misc = jnp.concatenate([
        jnp.concatenate([q1[D:], q1[D:]], axis=1),
        jnp.concatenate([q3[D:], q3[D:]], axis=1),
        jnp.concatenate([W3_b[None], z64], axis=1),
        jnp.concatenate([q5.T, z64], axis=1),
        jnp.concatenate([ln_g[None], z64], axis=1),
        jnp.concatenate([ln_b[None], z64], axis=1),
        jnp.concatenate([fc2_w, z64], axis=1),
        jnp.zeros((1, GW), _f32) + fc2_b[0],
    ], axis=0)
    misc = jnp.pad(misc, ((0, 8), (0, 0)))
    fc1bc = jnp.broadcast_to(fc1_b.reshape(D, 1), (D, 8))
    embt = emb_pad.T

    zdd = jnp.zeros((D, D), _f32)
    qbf = jnp.concatenate([
        jnp.concatenate([q1[:D], zdd], axis=1),
        jnp.concatenate([zdd, q1[:D]], axis=1)], axis=0)
    qbb = jnp.concatenate([
        jnp.concatenate([q3[:D], zdd], axis=1),
        jnp.concatenate([zdd, q3[:D]], axis=1)], axis=0)
    zc = jnp.zeros((D, 1), _f32)
    rf = jnp.concatenate([
        jnp.concatenate([q2, zc], axis=1),
        jnp.concatenate([zc, q2], axis=1)], axis=0)
    rf = jnp.pad(rf, ((0, 0), (0, 6)))
    rb = jnp.concatenate([
        jnp.concatenate([q4, zc], axis=1),
        jnp.concatenate([zc, q4], axis=1)], axis=0)
    rb = jnp.pad(rb, ((0, 0), (0, 6)))
    ones64 = jnp.ones((1, D), _f32)
    em = jnp.concatenate([
        jnp.concatenate([ones64, z64], axis=1),
        jnp.concatenate([z64, ones64], axis=1)], axis=0)
    em = jnp.pad(em, ((0, 6), (0, 0)))

    return _main(gathered, wf2, wb2, alias_re_inputs.astype(jnp.int32),
                 bias4, mats, misc, fc1bc, embt, qbf, qbb, rf, rb, em)
